# Initial kernel scaffold; baseline (speedup 1.0000x reference)
import functools

import jax
import jax.numpy as jnp
from jax import lax
from jax.experimental import pallas as pl
from jax.experimental.pallas import tpu as pltpu
from jax.experimental.pallas import tpu_sc as plsc

B = 16384
S = 200
V = 100000
C = 16

NC = 2
NS = 16
NW = NC * NS
BPW = B // NW
GROUP = 16
TOK = GROUP * S
IDXW = 128
NGATH = TOK // IDXW
NGROUP = BPW // GROUP


def _sc_body(idx_hbm, emb_hbm, bias_hbm, out_hbm, idx_v, rows_v, acc_v,
             bias_v, sem):
    wid = lax.axis_index("s") * NC + lax.axis_index("c")
    pltpu.sync_copy(bias_hbm, bias_v)
    bias_vec = bias_v[...]

    def group_body(g, _):
        idx_row0 = wid * (BPW * S // IDXW) + g * NGATH
        pltpu.sync_copy(idx_hbm.at[pl.ds(idx_row0, NGATH)], idx_v)
        copies = [
            pltpu.async_copy(
                emb_hbm.at[idx_v.at[c]],
                rows_v.at[pl.ds(c * IDXW, IDXW)],
                sem,
            )
            for c in range(NGATH)
        ]
        for cp in copies:
            cp.wait()

        def sample_body(i, _):
            base = i * S

            def red(k, accs):
                a0, a1, a2, a3 = accs
                o = base + k * 8
                a0 = a0 + rows_v[o + 0]
                a1 = a1 + rows_v[o + 1]
                a2 = a2 + rows_v[o + 2]
                a3 = a3 + rows_v[o + 3]
                a0 = a0 + rows_v[o + 4]
                a1 = a1 + rows_v[o + 5]
                a2 = a2 + rows_v[o + 6]
                a3 = a3 + rows_v[o + 7]
                return (a0, a1, a2, a3)

            z = jnp.zeros((16,), jnp.float32)
            a0, a1, a2, a3 = lax.fori_loop(0, S // 8, red, (z, z, z, z))
            acc_v[i] = ((a0 + a1) + (a2 + a3)) + bias_vec
            return 0

        lax.fori_loop(0, GROUP, sample_body, 0)
        pltpu.sync_copy(acc_v, out_hbm.at[pl.ds(wid * BPW + g * GROUP, GROUP)])
        return 0

    lax.fori_loop(0, NGROUP, group_body, 0)


_sc_pool = pl.kernel(
    _sc_body,
    out_type=jax.ShapeDtypeStruct((B, C), jnp.float32),
    mesh=plsc.VectorSubcoreMesh(
        core_axis_name="c", subcore_axis_name="s", num_cores=NC,
        num_subcores=NS),
    scratch_types=[
        pltpu.VMEM((NGATH, IDXW), jnp.int32),
        pltpu.VMEM((TOK, C), jnp.float32),
        pltpu.VMEM((GROUP, C), jnp.float32),
        pltpu.VMEM((C,), jnp.float32),
        pltpu.SemaphoreType.DMA,
    ],
)


def _tc_body(x_ref, o_ref):
    x = x_ref[...]
    m = jnp.max(x, axis=-1, keepdims=True)
    e = jnp.exp(x - m)
    lse = jnp.log(jnp.sum(e, axis=-1, keepdims=True))
    o_ref[...] = (x - m) - lse


_TCBLK = 2048
_tc_logsoftmax = pl.pallas_call(
    _tc_body,
    out_shape=jax.ShapeDtypeStruct((B, C), jnp.float32),
    grid=(B // _TCBLK,),
    in_specs=[pl.BlockSpec((_TCBLK, C), lambda i: (i, 0))],
    out_specs=pl.BlockSpec((_TCBLK, C), lambda i: (i, 0)),
)


def kernel(text, emb, bias):
    idx2d = text.reshape(B * S // IDXW, IDXW)
    logits = _sc_pool(idx2d, emb, bias)
    return _tc_logsoftmax(logits)


# speedup vs baseline: 40.6567x; 40.6567x over previous
import functools

import jax
import jax.numpy as jnp
from jax import lax
from jax.experimental import pallas as pl
from jax.experimental.pallas import tpu as pltpu
from jax.experimental.pallas import tpu_sc as plsc

B = 16384
S = 200
V = 100000
C = 16

NC = 2
NS = 16
NW = NC * NS
BPW = B // NW
GROUP = 16
TOK = GROUP * S
IDXW = 128
NGATH = TOK // IDXW
NGROUP = BPW // GROUP


def _sc_body(idx_hbm, emb_hbm, bias_hbm, out_hbm, idx_v, rows_v, acc_v,
             bias_v, sem):
    wid = lax.axis_index("s") * NC + lax.axis_index("c")
    pltpu.sync_copy(bias_hbm, bias_v)
    bias_vec = bias_v[...]

    def group_body(g, _):
        tok0 = wid * (BPW * S) + g * TOK
        pltpu.sync_copy(idx_hbm.at[pl.ds(tok0, TOK)], idx_v)
        copies = [
            pltpu.async_copy(
                emb_hbm.at[idx_v.at[pl.ds(c * IDXW, IDXW)]],
                rows_v.at[pl.ds(c * IDXW, IDXW)],
                sem,
            )
            for c in range(NGATH)
        ]
        for cp in copies:
            cp.wait()

        def sample_body(i, _):
            base = i * S

            def red(k, accs):
                a0, a1, a2, a3 = accs
                o = base + k * 8
                a0 = a0 + rows_v[o + 0]
                a1 = a1 + rows_v[o + 1]
                a2 = a2 + rows_v[o + 2]
                a3 = a3 + rows_v[o + 3]
                a0 = a0 + rows_v[o + 4]
                a1 = a1 + rows_v[o + 5]
                a2 = a2 + rows_v[o + 6]
                a3 = a3 + rows_v[o + 7]
                return (a0, a1, a2, a3)

            z = jnp.zeros((16,), jnp.float32)
            a0, a1, a2, a3 = lax.fori_loop(0, S // 8, red, (z, z, z, z))
            acc_v[i] = ((a0 + a1) + (a2 + a3)) + bias_vec
            return 0

        lax.fori_loop(0, GROUP, sample_body, 0)
        pltpu.sync_copy(acc_v, out_hbm.at[pl.ds(wid * BPW + g * GROUP, GROUP)])
        return 0

    lax.fori_loop(0, NGROUP, group_body, 0)


_sc_pool = pl.kernel(
    _sc_body,
    out_type=jax.ShapeDtypeStruct((B, C), jnp.float32),
    mesh=plsc.VectorSubcoreMesh(
        core_axis_name="c", subcore_axis_name="s", num_cores=NC,
        num_subcores=NS),
    scratch_types=[
        pltpu.VMEM((TOK,), jnp.int32),
        pltpu.VMEM((TOK, C), jnp.float32),
        pltpu.VMEM((GROUP, C), jnp.float32),
        pltpu.VMEM((C,), jnp.float32),
        pltpu.SemaphoreType.DMA,
    ],
    compiler_params=pltpu.CompilerParams(use_tc_tiling_on_sc=False),
)


def _tc_body(x_ref, o_ref):
    x = x_ref[...]
    m = jnp.max(x, axis=-1, keepdims=True)
    e = jnp.exp(x - m)
    lse = jnp.log(jnp.sum(e, axis=-1, keepdims=True))
    o_ref[...] = (x - m) - lse


_TCBLK = 2048
_tc_logsoftmax = pl.pallas_call(
    _tc_body,
    out_shape=jax.ShapeDtypeStruct((B, C), jnp.float32),
    grid=(B // _TCBLK,),
    in_specs=[pl.BlockSpec((_TCBLK, C), lambda i: (i, 0))],
    out_specs=pl.BlockSpec((_TCBLK, C), lambda i: (i, 0)),
)


def kernel(text, emb, bias):
    idx1d = text.reshape(B * S)
    logits = _sc_pool(idx1d, emb, bias)
    return _tc_logsoftmax(logits)


# speedup vs baseline: 55.8202x; 1.3730x over previous
import jax
import jax.numpy as jnp
from jax import lax
from jax.experimental import pallas as pl
from jax.experimental.pallas import tpu as pltpu
from jax.experimental.pallas import tpu_sc as plsc

B = 16384
S = 200
V = 100000
C = 16

NC = 2
NS = 16
NW = NC * NS
BPW = B // NW
GROUP = 16
TOK = GROUP * S
IDXW = 128
NGATH = TOK // IDXW
NGROUP = BPW // GROUP
NACC = 8


def _sc_body(idx_hbm, emb_hbm, bias_hbm, out_hbm, idx_v, rows_v, acc_v,
             bias_v, gsem0, gsem1, isem):
    wid = lax.axis_index("s") * NC + lax.axis_index("c")
    pltpu.sync_copy(bias_hbm, bias_v)
    bias_vec = bias_v[...]
    tile_tok0 = wid * (BPW * S)
    gsems = (gsem0, gsem1)

    def stage_and_fire(buf, g, sem):
        pltpu.sync_copy(idx_hbm.at[pl.ds(tile_tok0 + g * TOK, TOK)],
                        idx_v.at[buf])
        for c in range(NGATH):
            pltpu.async_copy(
                emb_hbm.at[idx_v.at[buf, pl.ds(c * IDXW, IDXW)]],
                rows_v.at[buf, pl.ds(c * IDXW, IDXW)],
                sem,
            )

    def drain(buf, sem):
        pltpu.make_async_copy(emb_hbm.at[pl.ds(0, TOK)], rows_v.at[buf],
                              sem).wait()

    def accumulate(buf, g):
        def sample_body(i, _):
            base = i * S
            a = [jnp.zeros((16,), jnp.float32) for _ in range(NACC)]
            for j in range(S):
                a[j % NACC] = a[j % NACC] + rows_v[buf, base + j]
            a = [a[0] + a[1], a[2] + a[3], a[4] + a[5], a[6] + a[7]]
            acc_v[i] = ((a[0] + a[1]) + (a[2] + a[3])) + bias_vec
            return 0

        lax.fori_loop(0, GROUP, sample_body, 0)
        pltpu.sync_copy(acc_v,
                        out_hbm.at[pl.ds(wid * BPW + g * GROUP, GROUP)])

    stage_and_fire(0, 0, gsem0)

    def pair_body(gg, _):
        g0 = 2 * gg
        stage_and_fire(1, g0 + 1, gsem1)
        drain(0, gsem0)
        accumulate(0, g0)

        @pl.when(gg != NGROUP // 2 - 1)
        def _():
            stage_and_fire(0, g0 + 2, gsem0)

        drain(1, gsem1)
        accumulate(1, g0 + 1)
        return 0

    lax.fori_loop(0, NGROUP // 2, pair_body, 0)


_sc_pool = pl.kernel(
    _sc_body,
    out_type=jax.ShapeDtypeStruct((B, C), jnp.float32),
    mesh=plsc.VectorSubcoreMesh(
        core_axis_name="c", subcore_axis_name="s", num_cores=NC,
        num_subcores=NS),
    scratch_types=[
        pltpu.VMEM((2, TOK), jnp.int32),
        pltpu.VMEM((2, TOK, C), jnp.float32),
        pltpu.VMEM((GROUP, C), jnp.float32),
        pltpu.VMEM((C,), jnp.float32),
        pltpu.SemaphoreType.DMA,
        pltpu.SemaphoreType.DMA,
        pltpu.SemaphoreType.DMA,
    ],
    compiler_params=pltpu.CompilerParams(use_tc_tiling_on_sc=False),
)


def _tc_body(x_ref, o_ref):
    x = x_ref[...]
    m = jnp.max(x, axis=-1, keepdims=True)
    e = jnp.exp(x - m)
    lse = jnp.log(jnp.sum(e, axis=-1, keepdims=True))
    o_ref[...] = (x - m) - lse


_TCBLK = 2048
_tc_logsoftmax = pl.pallas_call(
    _tc_body,
    out_shape=jax.ShapeDtypeStruct((B, C), jnp.float32),
    grid=(B // _TCBLK,),
    in_specs=[pl.BlockSpec((_TCBLK, C), lambda i: (i, 0))],
    out_specs=pl.BlockSpec((_TCBLK, C), lambda i: (i, 0)),
)


def kernel(text, emb, bias):
    idx1d = text.reshape(B * S)
    logits = _sc_pool(idx1d, emb, bias)
    return _tc_logsoftmax(logits)
